# Initial kernel scaffold; baseline (speedup 1.0000x reference)
import jax
import jax.numpy as jnp
from jax.experimental import pallas as pl


def kernel(x, timeoh, edge_index, Wm, bm, Wt, pool, node2vec, Wg, bg, Wq, bq, Wk, bk, Wv, bv, Wd, bd):
    raise NotImplementedError("write your pallas kernel here")


# speedup vs baseline: 8.7887x; 8.7887x over previous
import math

import jax
import jax.numpy as jnp
from jax import lax
from jax.experimental import pallas as pl
from jax.experimental.pallas import tpu as pltpu
from jax.experimental.pallas import tpu_sc as plsc

B, T, N, C = 8, 12, 325, 32
BT = B * T
NP = 384
M = 336
E = 5200
S6 = 192
NSTEP = E // 16
RB = T * M
R = B * RB


def _adj_attn_kernel(nv_ref, toh_ref, Wt_ref, pool_ref, xn_ref,
                     x1n_ref, x2n_ref, attn_ref):
    nv = nv_ref[...]
    z = jnp.dot(nv, nv.T, preferred_element_type=jnp.float32)
    z = jnp.maximum(z, 0.0)
    col = lax.broadcasted_iota(jnp.int32, (NP, NP), 1)
    z = jnp.where(col < N, z, -jnp.inf)
    m = jnp.max(z, axis=1, keepdims=True)
    ez = jnp.exp(z - m)
    s = jnp.sum(ez, axis=1, keepdims=True)
    A = ez / s
    row = lax.broadcasted_iota(jnp.int32, (NP, NP), 0)
    A = jnp.where(row < N, A, 0.0)
    Av = A[:M, :N]
    x1 = jnp.dot(Av, xn_ref[...], preferred_element_type=jnp.float32)
    x1n_ref[...] = x1
    x2n_ref[...] = jnp.dot(Av, x1[:N, :], preferred_element_type=jnp.float32)
    Wtp = jnp.dot(Wt_ref[...], pool_ref[...], preferred_element_type=jnp.float32)
    attn_ref[...] = jnp.dot(toh_ref[...], Wtp, preferred_element_type=jnp.float32)


def _proj_kernel(x_ref, x1_ref, x2_ref, attn_ref, Wm_ref, bm_ref, Wg_ref,
                 bg_ref, h_ref, aa_ref, xadp_ref):
    b = pl.program_id(0)
    xb = x_ref[...]
    x1 = x1_ref[...]
    x2 = x2_ref[...]
    h = jnp.dot(xb, Wm_ref[...], preferred_element_type=jnp.float32) + bm_ref[...]
    Wg = Wg_ref[...]
    xadp = (jnp.dot(xb, Wg[:C], preferred_element_type=jnp.float32)
            + jnp.dot(x1, Wg[C:2 * C], preferred_element_type=jnp.float32)
            + jnp.dot(x2, Wg[2 * C:], preferred_element_type=jnp.float32)
            + bg_ref[...])
    attn_b = attn_ref[pl.ds(b, 1), :]
    a_s = attn_b[:, :C]
    a_d = attn_b[:, C:]
    aa0 = lax.dot_general(h, a_s, (((1,), (1,)), ((), ())),
                          preferred_element_type=jnp.float32)
    aa1 = lax.dot_general(h, a_d, (((1,), (1,)), ((), ())),
                          preferred_element_type=jnp.float32)
    h_ref[...] = h
    aa_ref[...] = jnp.concatenate([aa0, aa1], axis=1)
    xadp_ref[...] = xadp


def _gat_sc_body(h_hbm, aa_hbm, s_hbm, d_hbm, y_hbm,
                 s_v, d_v, aa_v, ee_v, h_v,
                 hp0_v, hp1_v, hp2_v, hp3_v, rs0_v, rs1_v, rs2_v, rs3_v):
    hps = (hp0_v, hp1_v, hp2_v, hp3_v)
    nc = 2
    wid = lax.axis_index("s") * nc + lax.axis_index("c")
    pltpu.sync_copy(s_hbm, s_v)
    pltpu.sync_copy(d_hbm, d_v)
    for g in range(3):
        bt = wid * 3 + g
        pltpu.sync_copy(h_hbm.at[bt], h_v)
        pltpu.sync_copy(aa_hbm.at[bt], aa_v)

        def p1(i, mx):
            sv = s_v[pl.ds(i * 16, 16)]
            dv = d_v[pl.ds(i * 16, 16)]
            a = plsc.load_gather(aa_v, [sv * 2])
            bb = plsc.load_gather(aa_v, [dv * 2 + 1])
            v = a + bb
            v = jnp.where(v > 0, v, 0.01 * v)
            ee_v[pl.ds(i * 16, 16)] = v
            return jnp.maximum(mx, v)
        mx = lax.fori_loop(0, NSTEP, p1,
                           jnp.full((16,), -jnp.inf, jnp.float32))
        m = jnp.max(mx)

        def p2(i, c):
            v = ee_v[pl.ds(i * 16, 16)]
            ee_v[pl.ds(i * 16, 16)] = jnp.exp(v - m)
            return c
        lax.fori_loop(0, NSTEP, p2, 0)

        zv = jnp.zeros((16,), jnp.float32)
        def pz(i, c):
            hp0_v[pl.ds(i * 16, 16)] = zv
            hp1_v[pl.ds(i * 16, 16)] = zv
            hp2_v[pl.ds(i * 16, 16)] = zv
            hp3_v[pl.ds(i * 16, 16)] = zv
            return c
        lax.fori_loop(0, M * C // 16, pz, 0)

        def pz2(i, c):
            rs0_v[pl.ds(i * 16, 16)] = zv
            rs1_v[pl.ds(i * 16, 16)] = zv
            rs2_v[pl.ds(i * 16, 16)] = zv
            rs3_v[pl.ds(i * 16, 16)] = zv
            return c
        lax.fori_loop(0, 352 // 16, pz2, 0)

        lane0 = lax.iota(jnp.int32, 16) == 0
        rss = (rs0_v, rs1_v, rs2_v, rs3_v)
        def p3(i, c):
            base = i * 16
            sv = s_v[pl.ds(base, 16)]
            dv = d_v[pl.ds(base, 16)] * C
            svC = sv * C
            ev = ee_v[pl.ds(base, 16)]
            for k in range(16):
                hp = hps[k % 4]
                rs = rss[k % 4]
                se = svC[k]
                de = dv[k]
                w = ev[k]
                h0 = h_v[pl.ds(de, 16)]
                h1 = h_v[pl.ds(de + 16, 16)]
                hp[pl.ds(se, 16)] = hp[pl.ds(se, 16)] + w * h0
                hp[pl.ds(se + 16, 16)] = hp[pl.ds(se + 16, 16)] + w * h1
                sr = sv[k]
                wv = jnp.where(lane0, w, 0.0)
                rs[pl.ds(sr, 16)] = rs[pl.ds(sr, 16)] + wv
            return c
        lax.fori_loop(0, NSTEP, p3, 0)

        def p4(i, c):
            ds16 = pl.ds(i * 16, 16)
            rsv = rs0_v[ds16] + rs1_v[ds16] + rs2_v[ds16] + rs3_v[ds16]
            invv = 1.0 / (rsv + 9e-15)
            for k in range(16):
                n = (i * 16 + k) * C
                inv = invv[k]
                d0 = pl.ds(n, 16)
                d1 = pl.ds(n + 16, 16)
                r0 = (hp0_v[d0] + hp1_v[d0] + hp2_v[d0] + hp3_v[d0]) * inv
                r1 = (hp0_v[d1] + hp1_v[d1] + hp2_v[d1] + hp3_v[d1]) * inv
                hp0_v[d0] = jnp.where(r0 > 0, r0, jnp.exp(r0) - 1.0)
                hp0_v[d1] = jnp.where(r1 > 0, r1, jnp.exp(r1) - 1.0)
            return c
        lax.fori_loop(0, M // 16, p4, 0)

        pltpu.sync_copy(hp0_v, y_hbm.at[bt])


_gat_sc = pl.kernel(
    _gat_sc_body,
    mesh=plsc.VectorSubcoreMesh(core_axis_name="c", subcore_axis_name="s"),
    compiler_params=pltpu.CompilerParams(needs_layout_passes=False),
    out_type=jax.ShapeDtypeStruct((BT, M * C), jnp.float32),
    scratch_types=[
        pltpu.VMEM((E,), jnp.int32),
        pltpu.VMEM((E,), jnp.int32),
        pltpu.VMEM((M * 2,), jnp.float32),
        pltpu.VMEM((E,), jnp.float32),
        pltpu.VMEM((M * C,), jnp.float32),
        pltpu.VMEM((M * C,), jnp.float32),
        pltpu.VMEM((M * C,), jnp.float32),
        pltpu.VMEM((M * C,), jnp.float32),
        pltpu.VMEM((M * C,), jnp.float32),
        pltpu.VMEM((352,), jnp.float32),
        pltpu.VMEM((352,), jnp.float32),
        pltpu.VMEM((352,), jnp.float32),
        pltpu.VMEM((352,), jnp.float32),
    ],
)


P4 = 4
CL = C * P4


def _final_kernel(x_ref, y_ref, z_ref, Wq_ref, bq_ref, Wk_ref, bk_ref,
                  Wv_ref, bv_ref, Wd_ref, bd_ref, o_ref):
    inv_s = 1.0 / math.sqrt(6 * C)
    f32 = jnp.float32
    Wq = Wq_ref[...]
    Wk = Wk_ref[...]
    bq = bq_ref[...]
    bk = bk_ref[...]
    G = lax.dot_general(Wq, Wk, (((1,), (1,)), ((), ())),
                        preferred_element_type=f32) * inv_s
    aqc = jnp.dot(Wq, bk.T, preferred_element_type=f32) * inv_s
    ak = jnp.dot(bq, Wk.T, preferred_element_type=f32) * inv_s
    gamma = jnp.sum(bq * bk) * inv_s
    Wvd = jnp.dot(Wv_ref[...], Wd_ref[...], preferred_element_type=f32)
    cvd = jnp.dot(bv_ref[...], Wd_ref[...], preferred_element_type=f32) + bd_ref[...]

    zCC = jnp.zeros((C, C), f32)
    zC1 = jnp.zeros((C, 1), f32)
    z1C = jnp.zeros((1, C), f32)
    oC1 = jnp.ones((C, 1), f32)
    o1C = jnp.ones((1, C), f32)

    def bdiag(blk, zero):
        rows = [jnp.concatenate([blk if j == i else zero for j in range(P4)],
                                axis=1) for i in range(P4)]
        return jnp.concatenate(rows, axis=0)

    G4 = bdiag(G, zCC)
    Wvd4 = bdiag(Wvd, zCC)
    OB = bdiag(oC1, zC1)
    AQ = bdiag(aqc, zC1)
    SP = bdiag(o1C, z1C)
    ak4 = jnp.concatenate([ak] * P4, axis=1)
    cvd4 = jnp.concatenate([cvd] * P4, axis=1)

    o = (x_ref[...], y_ref[...], z_ref[...])
    U = [jnp.dot(oi, G4, preferred_element_type=f32) + ak4 for oi in o]
    al = [jnp.dot(oi, AQ, preferred_element_type=f32) + gamma for oi in o]
    sc = [[jnp.dot(U[i] * o[j], OB, preferred_element_type=f32) + al[i]
           for j in range(3)] for i in range(3)]
    w = [jnp.zeros_like(al[0]) for _ in range(3)]
    for i in range(3):
        mi = jnp.maximum(jnp.maximum(sc[i][0], sc[i][1]), sc[i][2])
        e0 = jnp.exp(sc[i][0] - mi)
        e1 = jnp.exp(sc[i][1] - mi)
        e2 = jnp.exp(sc[i][2] - mi)
        si = e0 + e1 + e2
        w[0] = w[0] + e0 / si
        w[1] = w[1] + e1 / si
        w[2] = w[2] + e2 / si
    blend = (jnp.dot(w[0], SP, preferred_element_type=f32) * o[0]
             + jnp.dot(w[1], SP, preferred_element_type=f32) * o[1]
             + jnp.dot(w[2], SP, preferred_element_type=f32) * o[2]) * (1.0 / 3.0)
    res = jnp.dot(blend, Wvd4, preferred_element_type=f32) + cvd4
    o_ref[...] = jnp.where(res > 0, res, jnp.exp(jnp.minimum(res, 0.0)) - 1.0)


def _full(shape):
    return pl.BlockSpec(shape, lambda i: tuple(0 for _ in shape))


def kernel(x, timeoh, edge_index, Wm, bm, Wt, pool, node2vec, Wg, bg,
           Wq, bq, Wk, bk, Wv, bv, Wd, bd):
    f32 = jnp.float32
    xr = x.reshape(BT, N, C)
    xn = jnp.transpose(xr, (1, 0, 2)).reshape(N, BT * C)
    nvp = jnp.pad(node2vec, ((0, NP - N), (0, 0)))

    x1n, x2n, attn = pl.pallas_call(
        _adj_attn_kernel,
        out_shape=[jax.ShapeDtypeStruct((M, BT * C), f32),
                   jax.ShapeDtypeStruct((M, BT * C), f32),
                   jax.ShapeDtypeStruct((B, 2 * C), f32)],
    )(nvp, timeoh, Wt, pool, xn)

    xp = jnp.pad(xr, ((0, 0), (0, M - N), (0, 0))).reshape(R, C)
    x1r = jnp.transpose(x1n.reshape(M, BT, C), (1, 0, 2)).reshape(R, C)
    x2r = jnp.transpose(x2n.reshape(M, BT, C), (1, 0, 2)).reshape(R, C)

    h, aa, xadp = pl.pallas_call(
        _proj_kernel,
        grid=(B,),
        in_specs=[
            pl.BlockSpec((RB, C), lambda i: (i, 0)),
            pl.BlockSpec((RB, C), lambda i: (i, 0)),
            pl.BlockSpec((RB, C), lambda i: (i, 0)),
            _full((B, 2 * C)),
            _full((C, C)),
            _full((1, C)),
            _full((3 * C, C)),
            _full((1, C)),
        ],
        out_specs=[
            pl.BlockSpec((RB, C), lambda i: (i, 0)),
            pl.BlockSpec((RB, 2), lambda i: (i, 0)),
            pl.BlockSpec((RB, C), lambda i: (i, 0)),
        ],
        out_shape=[jax.ShapeDtypeStruct((R, C), f32),
                   jax.ShapeDtypeStruct((R, 2), f32),
                   jax.ShapeDtypeStruct((R, C), f32)],
    )(xp, x1r, x2r, attn, Wm, bm.reshape(1, C), Wg, bg.reshape(1, C))

    y_flat = _gat_sc(h.reshape(BT, M * C), aa.reshape(BT, M * 2),
                     edge_index[0], edge_index[1])

    R4 = R // P4
    RB4 = RB // P4
    out_flat = pl.pallas_call(
        _final_kernel,
        grid=(B,),
        in_specs=[
            pl.BlockSpec((RB4, CL), lambda i: (i, 0)),
            pl.BlockSpec((RB4, CL), lambda i: (i, 0)),
            pl.BlockSpec((RB4, CL), lambda i: (i, 0)),
            _full((C, S6)),
            _full((1, S6)),
            _full((C, S6)),
            _full((1, S6)),
            _full((C, S6)),
            _full((1, S6)),
            _full((S6, C)),
            _full((1, C)),
        ],
        out_specs=pl.BlockSpec((RB4, CL), lambda i: (i, 0)),
        out_shape=jax.ShapeDtypeStruct((R4, CL), f32),
    )(xp.reshape(R4, CL), y_flat.reshape(R4, CL), xadp.reshape(R4, CL),
      Wq, bq.reshape(1, S6), Wk, bk.reshape(1, S6),
      Wv, bv.reshape(1, S6), Wd, bd.reshape(1, C))

    return out_flat.reshape(BT, M, C)[:, :N, :].reshape(B, T, N, C)


# speedup vs baseline: 9.0677x; 1.0317x over previous
import math

import jax
import jax.numpy as jnp
from jax import lax
from jax.experimental import pallas as pl
from jax.experimental.pallas import tpu as pltpu
from jax.experimental.pallas import tpu_sc as plsc

B, T, N, C = 8, 12, 325, 32
BT = B * T
NP = 384
M = 336
E = 5200
S6 = 192
NSTEP = E // 16
RB = T * M
R = B * RB


def _adj_attn_kernel(nv_ref, toh_ref, Wt_ref, pool_ref, xn_ref,
                     x1n_ref, x2n_ref, attn_ref):
    nv = nv_ref[...]
    z = jnp.dot(nv, nv.T, preferred_element_type=jnp.float32)
    z = jnp.maximum(z, 0.0)
    col = lax.broadcasted_iota(jnp.int32, (NP, NP), 1)
    z = jnp.where(col < N, z, -jnp.inf)
    m = jnp.max(z, axis=1, keepdims=True)
    ez = jnp.exp(z - m)
    s = jnp.sum(ez, axis=1, keepdims=True)
    A = ez / s
    row = lax.broadcasted_iota(jnp.int32, (NP, NP), 0)
    A = jnp.where(row < N, A, 0.0)
    Av = A[:M, :N]
    x1 = jnp.dot(Av, xn_ref[...], preferred_element_type=jnp.float32)
    x1n_ref[...] = x1
    x2n_ref[...] = jnp.dot(Av, x1[:N, :], preferred_element_type=jnp.float32)
    Wtp = jnp.dot(Wt_ref[...], pool_ref[...], preferred_element_type=jnp.float32)
    attn_ref[...] = jnp.dot(toh_ref[...], Wtp, preferred_element_type=jnp.float32)


def _proj_kernel(x_ref, x1_ref, x2_ref, attn_ref, Wm_ref, bm_ref, Wg_ref,
                 bg_ref, h_ref, aa_ref, xadp_ref):
    b = pl.program_id(0)
    xb = x_ref[...]
    x1 = x1_ref[...]
    x2 = x2_ref[...]
    h = jnp.dot(xb, Wm_ref[...], preferred_element_type=jnp.float32) + bm_ref[...]
    Wg = Wg_ref[...]
    xadp = (jnp.dot(xb, Wg[:C], preferred_element_type=jnp.float32)
            + jnp.dot(x1, Wg[C:2 * C], preferred_element_type=jnp.float32)
            + jnp.dot(x2, Wg[2 * C:], preferred_element_type=jnp.float32)
            + bg_ref[...])
    attn_b = attn_ref[pl.ds(b, 1), :]
    a_s = attn_b[:, :C]
    a_d = attn_b[:, C:]
    aa0 = lax.dot_general(h, a_s, (((1,), (1,)), ((), ())),
                          preferred_element_type=jnp.float32)
    aa1 = lax.dot_general(h, a_d, (((1,), (1,)), ((), ())),
                          preferred_element_type=jnp.float32)
    h_ref[...] = h
    aa_ref[...] = jnp.concatenate([aa0, aa1], axis=1)
    xadp_ref[...] = xadp


def _gat_sc_body(h_hbm, aa_hbm, s_hbm, d_hbm, y_hbm,
                 s_v, d_v, aa_v, ee_v, h_v,
                 hp0_v, hp1_v, hp2_v, hp3_v, rs0_v, rs1_v, rs2_v, rs3_v):
    hps = (hp0_v, hp1_v, hp2_v, hp3_v)
    nc = 2
    wid = lax.axis_index("s") * nc + lax.axis_index("c")
    pltpu.sync_copy(s_hbm, s_v)
    pltpu.sync_copy(d_hbm, d_v)
    for g in range(3):
        bt = wid * 3 + g
        pltpu.sync_copy(h_hbm.at[bt], h_v)
        pltpu.sync_copy(aa_hbm.at[bt], aa_v)

        def p1(i, mx):
            sv = s_v[pl.ds(i * 16, 16)]
            dv = d_v[pl.ds(i * 16, 16)]
            a = plsc.load_gather(aa_v, [sv * 2])
            bb = plsc.load_gather(aa_v, [dv * 2 + 1])
            v = a + bb
            v = jnp.where(v > 0, v, 0.01 * v)
            ee_v[pl.ds(i * 16, 16)] = v
            return jnp.maximum(mx, v)
        mx = lax.fori_loop(0, NSTEP, p1,
                           jnp.full((16,), -jnp.inf, jnp.float32))
        m = jnp.max(mx)

        zv = jnp.zeros((16,), jnp.float32)
        def pz(i, c):
            hp0_v[pl.ds(i * 16, 16)] = zv
            hp1_v[pl.ds(i * 16, 16)] = zv
            hp2_v[pl.ds(i * 16, 16)] = zv
            hp3_v[pl.ds(i * 16, 16)] = zv
            return c
        lax.fori_loop(0, M * C // 16, pz, 0)

        def pz2(i, c):
            rs0_v[pl.ds(i * 16, 16)] = zv
            rs1_v[pl.ds(i * 16, 16)] = zv
            rs2_v[pl.ds(i * 16, 16)] = zv
            rs3_v[pl.ds(i * 16, 16)] = zv
            return c
        lax.fori_loop(0, 352 // 16, pz2, 0)

        lane0 = lax.iota(jnp.int32, 16) == 0
        rss = (rs0_v, rs1_v, rs2_v, rs3_v)
        def p3(i, c):
            base = i * 16
            sv = s_v[pl.ds(base, 16)]
            dv = d_v[pl.ds(base, 16)] * C
            svC = sv * C
            ev = jnp.exp(ee_v[pl.ds(base, 16)] - m)
            for k in range(16):
                hp = hps[k % 4]
                rs = rss[k % 4]
                se = svC[k]
                de = dv[k]
                w = ev[k]
                h0 = h_v[pl.ds(de, 16)]
                h1 = h_v[pl.ds(de + 16, 16)]
                hp[pl.ds(se, 16)] = hp[pl.ds(se, 16)] + w * h0
                hp[pl.ds(se + 16, 16)] = hp[pl.ds(se + 16, 16)] + w * h1
                sr = sv[k]
                wv = jnp.where(lane0, w, 0.0)
                rs[pl.ds(sr, 16)] = rs[pl.ds(sr, 16)] + wv
            return c
        lax.fori_loop(0, NSTEP, p3, 0)

        def p4(i, c):
            ds16 = pl.ds(i * 16, 16)
            rsv = rs0_v[ds16] + rs1_v[ds16] + rs2_v[ds16] + rs3_v[ds16]
            invv = 1.0 / (rsv + 9e-15)
            for k in range(16):
                n = (i * 16 + k) * C
                inv = invv[k]
                d0 = pl.ds(n, 16)
                d1 = pl.ds(n + 16, 16)
                r0 = (hp0_v[d0] + hp1_v[d0] + hp2_v[d0] + hp3_v[d0]) * inv
                r1 = (hp0_v[d1] + hp1_v[d1] + hp2_v[d1] + hp3_v[d1]) * inv
                hp0_v[d0] = jnp.where(r0 > 0, r0, jnp.exp(r0) - 1.0)
                hp0_v[d1] = jnp.where(r1 > 0, r1, jnp.exp(r1) - 1.0)
            return c
        lax.fori_loop(0, M // 16, p4, 0)

        pltpu.sync_copy(hp0_v, y_hbm.at[bt])


_gat_sc = pl.kernel(
    _gat_sc_body,
    mesh=plsc.VectorSubcoreMesh(core_axis_name="c", subcore_axis_name="s"),
    compiler_params=pltpu.CompilerParams(needs_layout_passes=False),
    out_type=jax.ShapeDtypeStruct((BT, M * C), jnp.float32),
    scratch_types=[
        pltpu.VMEM((E,), jnp.int32),
        pltpu.VMEM((E,), jnp.int32),
        pltpu.VMEM((M * 2,), jnp.float32),
        pltpu.VMEM((E,), jnp.float32),
        pltpu.VMEM((M * C,), jnp.float32),
        pltpu.VMEM((M * C,), jnp.float32),
        pltpu.VMEM((M * C,), jnp.float32),
        pltpu.VMEM((M * C,), jnp.float32),
        pltpu.VMEM((M * C,), jnp.float32),
        pltpu.VMEM((352,), jnp.float32),
        pltpu.VMEM((352,), jnp.float32),
        pltpu.VMEM((352,), jnp.float32),
        pltpu.VMEM((352,), jnp.float32),
    ],
)


P4 = 4
CL = C * P4


def _final_kernel(x_ref, y_ref, z_ref, Wq_ref, bq_ref, Wk_ref, bk_ref,
                  Wv_ref, bv_ref, Wd_ref, bd_ref, o_ref):
    inv_s = 1.0 / math.sqrt(6 * C)
    f32 = jnp.float32
    Wq = Wq_ref[...]
    Wk = Wk_ref[...]
    bq = bq_ref[...]
    bk = bk_ref[...]
    G = lax.dot_general(Wq, Wk, (((1,), (1,)), ((), ())),
                        preferred_element_type=f32) * inv_s
    aqc = jnp.dot(Wq, bk.T, preferred_element_type=f32) * inv_s
    ak = jnp.dot(bq, Wk.T, preferred_element_type=f32) * inv_s
    gamma = jnp.sum(bq * bk) * inv_s
    Wvd = jnp.dot(Wv_ref[...], Wd_ref[...], preferred_element_type=f32)
    cvd = jnp.dot(bv_ref[...], Wd_ref[...], preferred_element_type=f32) + bd_ref[...]

    zCC = jnp.zeros((C, C), f32)
    zC1 = jnp.zeros((C, 1), f32)
    z1C = jnp.zeros((1, C), f32)
    oC1 = jnp.ones((C, 1), f32)
    o1C = jnp.ones((1, C), f32)

    def bdiag(blk, zero):
        rows = [jnp.concatenate([blk if j == i else zero for j in range(P4)],
                                axis=1) for i in range(P4)]
        return jnp.concatenate(rows, axis=0)

    G4 = bdiag(G, zCC)
    Wvd4 = bdiag(Wvd, zCC)
    OB = bdiag(oC1, zC1)
    AQ = bdiag(aqc, zC1)
    SP = bdiag(o1C, z1C)
    ak4 = jnp.concatenate([ak] * P4, axis=1)
    cvd4 = jnp.concatenate([cvd] * P4, axis=1)

    o = (x_ref[...], y_ref[...], z_ref[...])
    U = [jnp.dot(oi, G4, preferred_element_type=f32) + ak4 for oi in o]
    al = [jnp.dot(oi, AQ, preferred_element_type=f32) + gamma for oi in o]
    sc = [[jnp.dot(U[i] * o[j], OB, preferred_element_type=f32) + al[i]
           for j in range(3)] for i in range(3)]
    w = [jnp.zeros_like(al[0]) for _ in range(3)]
    for i in range(3):
        mi = jnp.maximum(jnp.maximum(sc[i][0], sc[i][1]), sc[i][2])
        e0 = jnp.exp(sc[i][0] - mi)
        e1 = jnp.exp(sc[i][1] - mi)
        e2 = jnp.exp(sc[i][2] - mi)
        si = e0 + e1 + e2
        w[0] = w[0] + e0 / si
        w[1] = w[1] + e1 / si
        w[2] = w[2] + e2 / si
    blend = (jnp.dot(w[0], SP, preferred_element_type=f32) * o[0]
             + jnp.dot(w[1], SP, preferred_element_type=f32) * o[1]
             + jnp.dot(w[2], SP, preferred_element_type=f32) * o[2]) * (1.0 / 3.0)
    res = jnp.dot(blend, Wvd4, preferred_element_type=f32) + cvd4
    o_ref[...] = jnp.where(res > 0, res, jnp.exp(jnp.minimum(res, 0.0)) - 1.0)


def _full(shape):
    return pl.BlockSpec(shape, lambda i: tuple(0 for _ in shape))


def kernel(x, timeoh, edge_index, Wm, bm, Wt, pool, node2vec, Wg, bg,
           Wq, bq, Wk, bk, Wv, bv, Wd, bd):
    f32 = jnp.float32
    xr = x.reshape(BT, N, C)
    xn = jnp.transpose(xr, (1, 0, 2)).reshape(N, BT * C)
    nvp = jnp.pad(node2vec, ((0, NP - N), (0, 0)))

    x1n, x2n, attn = pl.pallas_call(
        _adj_attn_kernel,
        out_shape=[jax.ShapeDtypeStruct((M, BT * C), f32),
                   jax.ShapeDtypeStruct((M, BT * C), f32),
                   jax.ShapeDtypeStruct((B, 2 * C), f32)],
    )(nvp, timeoh, Wt, pool, xn)

    xp = jnp.pad(xr, ((0, 0), (0, M - N), (0, 0))).reshape(R, C)
    x1r = jnp.transpose(x1n.reshape(M, BT, C), (1, 0, 2)).reshape(R, C)
    x2r = jnp.transpose(x2n.reshape(M, BT, C), (1, 0, 2)).reshape(R, C)

    h, aa, xadp = pl.pallas_call(
        _proj_kernel,
        grid=(B,),
        in_specs=[
            pl.BlockSpec((RB, C), lambda i: (i, 0)),
            pl.BlockSpec((RB, C), lambda i: (i, 0)),
            pl.BlockSpec((RB, C), lambda i: (i, 0)),
            _full((B, 2 * C)),
            _full((C, C)),
            _full((1, C)),
            _full((3 * C, C)),
            _full((1, C)),
        ],
        out_specs=[
            pl.BlockSpec((RB, C), lambda i: (i, 0)),
            pl.BlockSpec((RB, 2), lambda i: (i, 0)),
            pl.BlockSpec((RB, C), lambda i: (i, 0)),
        ],
        out_shape=[jax.ShapeDtypeStruct((R, C), f32),
                   jax.ShapeDtypeStruct((R, 2), f32),
                   jax.ShapeDtypeStruct((R, C), f32)],
    )(xp, x1r, x2r, attn, Wm, bm.reshape(1, C), Wg, bg.reshape(1, C))

    y_flat = _gat_sc(h.reshape(BT, M * C), aa.reshape(BT, M * 2),
                     edge_index[0], edge_index[1])

    R4 = R // P4
    RB4 = RB // P4
    out_flat = pl.pallas_call(
        _final_kernel,
        grid=(B,),
        in_specs=[
            pl.BlockSpec((RB4, CL), lambda i: (i, 0)),
            pl.BlockSpec((RB4, CL), lambda i: (i, 0)),
            pl.BlockSpec((RB4, CL), lambda i: (i, 0)),
            _full((C, S6)),
            _full((1, S6)),
            _full((C, S6)),
            _full((1, S6)),
            _full((C, S6)),
            _full((1, S6)),
            _full((S6, C)),
            _full((1, C)),
        ],
        out_specs=pl.BlockSpec((RB4, CL), lambda i: (i, 0)),
        out_shape=jax.ShapeDtypeStruct((R4, CL), f32),
    )(xp.reshape(R4, CL), y_flat.reshape(R4, CL), xadp.reshape(R4, CL),
      Wq, bq.reshape(1, S6), Wk, bk.reshape(1, S6),
      Wv, bv.reshape(1, S6), Wd, bd.reshape(1, C))

    return out_flat.reshape(BT, M, C)[:, :N, :].reshape(B, T, N, C)
